# baseline (speedup 1.0000x reference)
import functools

import jax
import jax.numpy as jnp
from jax import lax
from jax.experimental import pallas as pl
from jax.experimental.pallas import tpu as pltpu
from jax.experimental.pallas import tpu_sc as plsc

N = 10000
E = 320000
IN_DIM = 128
H = 32
G = 256
DEC = 16

NW = 32
RPS = 632
AGN = 16 * RPS
NPAD = AGN
BLK = RPS
NB = 16
EPT = 10240
EPAD = NW * EPT

CH = 128
K = 8

NCH32 = EPT // CH
NG32 = NCH32 // K

EPT1 = EPAD // 16
NCH1 = EPT1 // CH
NG1 = NCH1 // K


def _pipelined_scatter(y_hbm, src_v, dst_v, rows_v, agg_sh, sem_g, sem_s, ng):
    for j in range(K):
        pltpu.async_copy(y_hbm.at[src_v.at[j]], rows_v.at[0, j], sem_g)

    def _drain(sem, half, j):
        pltpu.make_async_copy(y_hbm.at[pl.ds(0, CH)],
                              rows_v.at[half, j], sem).wait()

    def _group(g, carry):
        half = lax.rem(g, 2)
        other = 1 - half
        for j in range(K):
            _drain(sem_g, half, j)

        @pl.when(g >= 1)
        def _():
            for j in range(K):
                _drain(sem_s, other, j)

        @pl.when(g + 1 < ng)
        def _():
            for j in range(K):
                pltpu.async_copy(y_hbm.at[src_v.at[(g + 1) * K + j]],
                                 rows_v.at[other, j], sem_g)

        for j in range(K):
            pltpu.async_copy(rows_v.at[half, j],
                             agg_sh.at[dst_v.at[g * K + j]], sem_s, add=True)
        return carry

    lax.fori_loop(0, ng, _group, 0)
    for j in range(K):
        _drain(sem_s, (ng - 1) % 2, j)


@functools.partial(
    pl.kernel,
    out_type=jax.ShapeDtypeStruct((2 * AGN, H), jnp.float32),
    mesh=plsc.VectorSubcoreMesh(core_axis_name="c", subcore_axis_name="s"),
    compiler_params=pltpu.CompilerParams(use_tc_tiling_on_sc=False),
    scratch_types=[
        pltpu.VMEM((NCH32, CH), jnp.int32),
        pltpu.VMEM((NCH32, CH), jnp.int32),
        pltpu.VMEM((2, K, CH, H), jnp.float32),
        pltpu.VMEM_SHARED((AGN, H), jnp.float32),
        pltpu.SemaphoreType.DMA,
        pltpu.SemaphoreType.DMA,
    ],
)
def _sc_agg32(y_hbm, srcp_hbm, dstp_hbm, z_hbm, out_hbm,
              src_v, dst_v, rows_v, agg_sh, sem_g, sem_s):
    c = lax.axis_index("c")
    s = lax.axis_index("s")
    tid = c * 16 + s
    pltpu.sync_copy(z_hbm, agg_sh.at[pl.ds(s * RPS, RPS)])
    pltpu.sync_copy(srcp_hbm.at[pl.ds(tid * NCH32, NCH32)], src_v)
    pltpu.sync_copy(dstp_hbm.at[pl.ds(tid * NCH32, NCH32)], dst_v)
    plsc.subcore_barrier()
    _pipelined_scatter(y_hbm, src_v, dst_v, rows_v, agg_sh,
                       sem_g, sem_s, NG32)
    plsc.subcore_barrier()
    pltpu.sync_copy(agg_sh.at[pl.ds(s * RPS, RPS)],
                    out_hbm.at[pl.ds(c * AGN + s * RPS, RPS)])


@functools.partial(
    pl.kernel,
    out_type=jax.ShapeDtypeStruct((4 * AGN, H), jnp.float32),
    mesh=plsc.VectorSubcoreMesh(core_axis_name="c", subcore_axis_name="s"),
    compiler_params=pltpu.CompilerParams(use_tc_tiling_on_sc=False),
    scratch_types=[
        pltpu.VMEM((NCH1, CH), jnp.int32),
        pltpu.VMEM((NCH1, CH), jnp.int32),
        pltpu.VMEM((2, K, CH, H), jnp.float32),
        pltpu.VMEM_SHARED((AGN, H), jnp.float32),
        pltpu.SemaphoreType.DMA,
        pltpu.SemaphoreType.DMA,
    ],
)
def _sc_agg128(x4_hbm, srcp4_hbm, dstp1_hbm, z_hbm, out_hbm,
               src_v, dst_v, rows_v, agg_sh, sem_g, sem_s):
    c = lax.axis_index("c")
    s = lax.axis_index("s")
    pltpu.sync_copy(dstp1_hbm.at[pl.ds(s * NCH1, NCH1)], dst_v)
    for qq in range(2):
        q = c * 2 + qq
        pltpu.sync_copy(z_hbm, agg_sh.at[pl.ds(s * RPS, RPS)])
        pltpu.sync_copy(srcp4_hbm.at[pl.ds((q * 16 + s) * NCH1, NCH1)], src_v)
        plsc.subcore_barrier()
        _pipelined_scatter(x4_hbm, src_v, dst_v, rows_v, agg_sh,
                           sem_g, sem_s, NG1)
        plsc.subcore_barrier()
        pltpu.sync_copy(agg_sh.at[pl.ds(s * RPS, RPS)],
                        out_hbm.at[pl.ds(q * AGN + s * RPS, RPS)])


def _gin_mlp(hin, w1_ref, b1_ref, w2_ref, b2_ref, s_ref, sb_ref, o_ref):
    t = jnp.maximum(jnp.dot(hin, w1_ref[...],
                            preferred_element_type=jnp.float32)
                    + b1_ref[...], 0.0)
    u = jnp.dot(t, w2_ref[...], preferred_element_type=jnp.float32) \
        + b2_ref[...]
    o_ref[0:N, :] = u * s_ref[...] + sb_ref[...]
    o_ref[N:NPAD, :] = jnp.zeros((NPAD - N, H), jnp.float32)


def _layer_body(h_ref, a_ref, w1_ref, b1_ref, w2_ref, b2_ref,
                s_ref, sb_ref, o_ref):
    hin = h_ref[0:N, :] + (a_ref[0:N, :] + a_ref[AGN:AGN + N, :])
    _gin_mlp(hin, w1_ref, b1_ref, w2_ref, b2_ref, s_ref, sb_ref, o_ref)


def _layer(h, a, w1, b1, w2, b2, s, sb):
    return pl.pallas_call(
        _layer_body,
        out_shape=jax.ShapeDtypeStruct((NPAD, H), jnp.float32),
    )(h, a, w1, b1, w2, b2, s, sb)


def _layer1_body(x_ref, a_ref, w1_ref, b1_ref, w2_ref, b2_ref,
                 s_ref, sb_ref, o_ref):
    agg = jnp.concatenate(
        [a_ref[q * AGN:q * AGN + N, :] for q in range(4)], axis=1)
    hin = x_ref[...] + agg
    _gin_mlp(hin, w1_ref, b1_ref, w2_ref, b2_ref, s_ref, sb_ref, o_ref)


def _layer1(x, a, w1, b1, w2, b2, s, sb):
    return pl.pallas_call(
        _layer1_body,
        out_shape=jax.ShapeDtypeStruct((NPAD, H), jnp.float32),
    )(x, a, w1, b1, w2, b2, s, sb)


def _tail_body(h_ref, bat_ref, wb_ref, bb_ref, wm_ref, bm_ref,
               o_ref, sums_s, cnt_s):
    j = pl.program_id(0)

    @pl.when(j == 0)
    def _():
        sums_s[...] = jnp.zeros_like(sums_s)
        cnt_s[...] = jnp.zeros_like(cnt_s)

    @pl.when(j < NB)
    def _():
        bat = bat_ref[...].reshape(1, BLK)
        oh = (lax.broadcasted_iota(jnp.int32, (G, BLK), 0) == bat
              ).astype(jnp.float32)
        sums_s[...] = sums_s[...] + jnp.dot(
            oh, h_ref[...], preferred_element_type=jnp.float32,
            precision=jax.lax.Precision.HIGHEST)
        cnt_s[...] = cnt_s[...] + jnp.sum(oh, axis=1, keepdims=True)

    @pl.when(j == NB)
    def _():
        cnt = cnt_s[...]
        emb = jnp.where(cnt > 0.0, sums_s[...] / jnp.maximum(cnt, 1.0), 0.0)
        z = jnp.maximum(
            jnp.dot(emb, wb_ref[...],
                    preferred_element_type=jnp.float32) + bb_ref[...], 0.0)
        logit = jnp.dot(z, wm_ref[...],
                        preferred_element_type=jnp.float32) + bm_ref[...]
        o_ref[...] = 1.0 / (1.0 + jnp.exp(-logit))


def _tail(h, bat3, wb, bb, wm, bm):
    idx = lambda j: (jnp.minimum(j, NB - 1), 0)
    return pl.pallas_call(
        _tail_body,
        grid=(NB + 1,),
        in_specs=[
            pl.BlockSpec((BLK, H), idx),
            pl.BlockSpec((1, 1, BLK), lambda j: (jnp.minimum(j, NB - 1), 0, 0)),
            pl.BlockSpec((H, DEC), lambda j: (0, 0)),
            pl.BlockSpec((1, DEC), lambda j: (0, 0)),
            pl.BlockSpec((DEC, 1), lambda j: (0, 0)),
            pl.BlockSpec((1, 1), lambda j: (0, 0)),
        ],
        out_specs=pl.BlockSpec((G, 1), lambda j: (0, 0)),
        scratch_shapes=[
            pltpu.VMEM((G, H), jnp.float32),
            pltpu.VMEM((G, 1), jnp.float32),
        ],
        out_shape=jax.ShapeDtypeStruct((G, 1), jnp.float32),
    )(h, bat3, wb, bb, wm, bm)


def kernel(x, edge_index, batch,
           w1_1, b1_1, w1_2, b1_2, bn1_g, bn1_b, bn1_m, bn1_v,
           w2_1, b2_1, w2_2, b2_2, bn2_g, bn2_b, bn2_m, bn2_v,
           w3_1, b3_1, w3_2, b3_2, bn3_g, bn3_b, bn3_m, bn3_v,
           wb, bb, wm, bm):
    s1 = bn1_g * lax.rsqrt(bn1_v + 1e-5)
    s2 = bn2_g * lax.rsqrt(bn2_v + 1e-5)
    s3 = bn3_g * lax.rsqrt(bn3_v + 1e-5)
    sb1 = (bn1_b - bn1_m * s1).reshape(1, H)
    sb2 = (bn2_b - bn2_m * s2).reshape(1, H)
    sb3 = (bn3_b - bn3_m * s3).reshape(1, H)

    pad = EPAD - E
    src = jnp.concatenate([edge_index[0], jnp.zeros((pad,), jnp.int32)])
    dst = jnp.concatenate([edge_index[1], jnp.full((pad,), N, jnp.int32)])
    srcp32 = src.reshape(NW * NCH32, CH)
    dstp32 = dst.reshape(NW * NCH32, CH)
    x4 = x.reshape(N, 4, H).transpose(1, 0, 2).reshape(4 * N, H)
    srct = src.reshape(16, NCH1, CH)
    srcp4 = jnp.concatenate([srct + q * N for q in range(4)],
                            axis=0).reshape(4 * 16 * NCH1, CH)
    dstp1 = dst.reshape(16 * NCH1, CH)
    z32 = jnp.zeros((RPS, H), jnp.float32)
    bat3 = jnp.concatenate(
        [batch, jnp.full((NPAD - N,), G, jnp.int32)]).reshape(NB, 1, BLK)

    a1 = _sc_agg128(x4, srcp4, dstp1, z32)
    h1 = _layer1(x, a1, w1_1, b1_1.reshape(1, H), w1_2, b1_2.reshape(1, H),
                 s1.reshape(1, H), sb1)
    a2 = _sc_agg32(h1, srcp32, dstp32, z32)
    h2 = _layer(h1, a2, w2_1, b2_1.reshape(1, H), w2_2, b2_2.reshape(1, H),
                s2.reshape(1, H), sb2)
    a3 = _sc_agg32(h2, srcp32, dstp32, z32)
    h3 = _layer(h2, a3, w3_1, b3_1.reshape(1, H), w3_2, b3_2.reshape(1, H),
                s3.reshape(1, H), sb3)
    return _tail(h3, bat3, wb, bb.reshape(1, DEC), wm, bm.reshape(1, 1))
